# Initial kernel scaffold; baseline (speedup 1.0000x reference)
import jax
import jax.numpy as jnp
from jax.experimental import pallas as pl


def kernel(pos, mesh_pos, velocities, inv_mass, node_attr, mask, params, edge_index, world_edge_index, batch, face_index):
    raise NotImplementedError("write your pallas kernel here")


# speedup vs baseline: 1.3184x; 1.3184x over previous
import functools

import jax
import jax.numpy as jnp
from jax.experimental import pallas as pl
from jax.experimental.pallas import tpu as pltpu

_pc = pl.pallas_call

HID = 128
BE = 2000


def _mm(a, b):
    return jax.lax.dot_general(a, b, (((1,), (0,)), ((), ())),
                               preferred_element_type=jnp.float32)


def _ln(h, g, b):
    m = jnp.mean(h, axis=1, keepdims=True)
    v = jnp.mean((h - m) ** 2, axis=1, keepdims=True)
    return (h - m) / jnp.sqrt(v + 1e-5) * g + b


def _row_spec(w=HID):
    return pl.BlockSpec((BE, w), lambda i: (i, 0))


def _full_spec(shape):
    return pl.BlockSpec(shape, lambda i: tuple(0 for _ in shape))


def _edge_upd_body(ga, gb, e, w1c, w2, w3, b2, b3, lng, lnb, out):
    h = jnp.maximum(ga[:] + gb[:] + _mm(e[:], w1c[:]), 0.0)
    h = jnp.maximum(_mm(h, w2[:]) + b2[:], 0.0)
    h = _mm(h, w3[:]) + b3[:]
    out[:] = e[:] + _ln(h, lng[:], lnb[:])


def _edge_update(ga, gb, e, w1c, w2, w3, b2, b3, lng, lnb):
    n = e.shape[0]
    return _pc(
        _edge_upd_body,
        grid=(n // BE,),
        in_specs=[_row_spec(), _row_spec(), _row_spec(),
                  _full_spec((HID, HID)), _full_spec((HID, HID)), _full_spec((HID, HID)),
                  _full_spec((1, HID)), _full_spec((1, HID)),
                  _full_spec((1, HID)), _full_spec((1, HID))],
        out_specs=_row_spec(),
        out_shape=jax.ShapeDtypeStruct((n, HID), jnp.float32),
    )(ga, gb, e, w1c, w2, w3, b2, b3, lng, lnb)


def _node_upd_body(x, am0, am1, aw0, aw1, wx, wm, ww, b1, w2, w3, b2, b3,
                   lng, lnb, out):
    xx = x[:]
    h = (_mm(xx, wx[:]) + _mm(am0[:] + am1[:], wm[:])
         + _mm(aw0[:] + aw1[:], ww[:]) + b1[:])
    h = jnp.maximum(h, 0.0)
    h = jnp.maximum(_mm(h, w2[:]) + b2[:], 0.0)
    h = _mm(h, w3[:]) + b3[:]
    out[:] = xx + _ln(h, lng[:], lnb[:])


def _node_update(x, am0, am1, aw0, aw1, wx, wm, ww, b1, w2, w3, b2, b3, lng, lnb):
    n = x.shape[0]
    return _pc(
        _node_upd_body,
        grid=(n // BE,),
        in_specs=[_row_spec()] * 5
        + [_full_spec((HID, HID))] * 3 + [_full_spec((1, HID))]
        + [_full_spec((HID, HID))] * 2 + [_full_spec((1, HID))] * 4,
        out_specs=_row_spec(),
        out_shape=jax.ShapeDtypeStruct((n, HID), jnp.float32),
    )(x, am0, am1, aw0, aw1, wx, wm, ww, b1, w2, w3, b2, b3, lng, lnb)


def _project_body(x, wam, wbm, waw, wbw, b1m, b1w, oam, obm, oaw, obw):
    xx = x[:]
    oam[:] = _mm(xx, wam[:]) + b1m[:]
    obm[:] = _mm(xx, wbm[:])
    oaw[:] = _mm(xx, waw[:]) + b1w[:]
    obw[:] = _mm(xx, wbw[:])


def _project(x, wam, wbm, waw, wbw, b1m, b1w):
    n = x.shape[0]
    sd = jax.ShapeDtypeStruct((n, HID), jnp.float32)
    return _pc(
        _project_body,
        grid=(n // BE,),
        in_specs=[_row_spec()] + [_full_spec((HID, HID))] * 4
        + [_full_spec((1, HID))] * 2,
        out_specs=[_row_spec()] * 4,
        out_shape=[sd, sd, sd, sd],
    )(x, wam, wbm, waw, wbw, b1m, b1w)


def _mlp_body(ln, xin, w1, w2, w3, b1, b2, b3, lng, lnb, out):
    h = jnp.maximum(_mm(xin[:], w1[:]) + b1[:], 0.0)
    h = jnp.maximum(_mm(h, w2[:]) + b2[:], 0.0)
    h = _mm(h, w3[:]) + b3[:]
    out[:] = _ln(h, lng[:], lnb[:]) if ln else h


def _mlp_call(xin, w1, w2, w3, b1, b2, b3, lng, lnb, ln=True):
    n, k = xin.shape
    return _pc(
        functools.partial(_mlp_body, ln),
        grid=(n // BE,),
        in_specs=[_row_spec(k), _full_spec((k, HID)),
                  _full_spec((HID, HID)), _full_spec((HID, HID)),
                  _full_spec((1, HID)), _full_spec((1, HID)), _full_spec((1, HID)),
                  _full_spec((1, HID)), _full_spec((1, HID))],
        out_specs=_row_spec(),
        out_shape=jax.ShapeDtypeStruct((n, HID), jnp.float32),
    )(xin, w1, w2, w3, b1, b2, b3, lng, lnb)


def _feat_body(world, ts_ref, td_ref, raw, stats):
    i = pl.program_id(0)
    ts, td = ts_ref[:], td_ref[:]
    rel = ts[:, 0:2] - td[:, 0:2]
    rel_vel = ts[:, 4:6] - td[:, 4:6]
    if world:
        r = jnp.concatenate([rel, rel_vel, jnp.zeros_like(rel),
                             jnp.zeros_like(rel)], axis=1)
    else:
        dist = jnp.sqrt(jnp.sum(rel * rel, axis=1, keepdims=True))
        rel_s = ts[:, 2:4] - td[:, 2:4]
        dist_s = jnp.sqrt(jnp.sum(rel_s * rel_s, axis=1, keepdims=True))
        strain = (dist - dist_s) / (dist_s + 1e-6)
        strain_vec = strain * (rel / dist + 1e-6)
        r = jnp.concatenate([rel, strain_vec, rel_vel, dist, dist_s], axis=1)
    raw[:] = r
    st = jnp.concatenate([jnp.sum(r, axis=0, keepdims=True),
                          jnp.sum(r * r, axis=0, keepdims=True),
                          jnp.zeros((6, 8), jnp.float32)], axis=0)

    @pl.when(i == 0)
    def _():
        stats[:] = st

    @pl.when(i > 0)
    def _():
        stats[:] = stats[:] + st


def _edge_feats(ts, td, world):
    n = ts.shape[0]
    return _pc(
        functools.partial(_feat_body, world),
        grid=(n // BE,),
        in_specs=[_row_spec(8), _row_spec(8)],
        out_specs=[_row_spec(8), _full_spec((8, 8))],
        out_shape=[jax.ShapeDtypeStruct((n, 8), jnp.float32),
                   jax.ShapeDtypeStruct((8, 8), jnp.float32)],
    )(ts, td)


def _enc_body(count, raw, stats, w1, w2, w3, b1, b2, b3, lng, lnb, out):
    st = stats[:]
    m = st[0:1, :] * (1.0 / count)
    v = st[1:2, :] * (1.0 / count) - m * m
    norm = (raw[:] - m) / jnp.sqrt(v + 1e-8)
    h = jnp.maximum(_mm(norm, w1[:]) + b1[:], 0.0)
    h = jnp.maximum(_mm(h, w2[:]) + b2[:], 0.0)
    h = _mm(h, w3[:]) + b3[:]
    out[:] = _ln(h, lng[:], lnb[:])


def _encode_edges(raw, stats, w1, w2, w3, b1, b2, b3, lng, lnb):
    n = raw.shape[0]
    return _pc(
        functools.partial(_enc_body, float(n)),
        grid=(n // BE,),
        in_specs=[_row_spec(8), _full_spec((8, 8)), _full_spec((8, HID)),
                  _full_spec((HID, HID)), _full_spec((HID, HID)),
                  _full_spec((1, HID)), _full_spec((1, HID)), _full_spec((1, HID)),
                  _full_spec((1, HID)), _full_spec((1, HID))],
        out_specs=_row_spec(),
        out_shape=jax.ShapeDtypeStruct((n, HID), jnp.float32),
    )(raw, stats, w1, w2, w3, b1, b2, b3, lng, lnb)


def _vrow(v):
    return v.reshape(1, HID)


def _symlog(x):
    return jnp.sign(x) * jnp.log1p(jnp.abs(x))


def _znorm_masked(x, mask):
    s = jnp.sum(mask) + 1e-8
    m = jnp.sum(x * mask, axis=0) / s
    v = jnp.sum(((x - m) ** 2) * mask, axis=0) / s
    return (x - m) / jnp.sqrt(v + 1e-8)


def _gather_rows(table, idx):
    return table[idx]


def _scatter_add(vals, idx, n):
    z = jnp.zeros((n, HID), vals.dtype)
    return z.at[idx].add(vals), jnp.zeros((n, HID), vals.dtype)


def kernel(pos, mesh_pos, velocities, inv_mass, node_attr, mask, params,
           edge_index, world_edge_index, batch, face_index):
    n = pos.shape[0]
    src, dst = edge_index[0], edge_index[1]
    sw, dw = world_edge_index[0], world_edge_index[1]
    curr_vel = velocities[:, 2:]

    norm_vel = _znorm_masked(velocities, mask)
    inv_m = _znorm_masked(_symlog(inv_mass), mask)
    x9 = jnp.concatenate([norm_vel, inv_m, node_attr], axis=1)
    x16 = jnp.pad(x9, ((0, 0), (0, 7)))

    enc = params["node_enc"]
    w1 = jnp.pad(enc["ws"][0], ((0, 7), (0, 0)))
    x = _mlp_call(x16, w1, enc["ws"][1], enc["ws"][2], _vrow(enc["bs"][0]),
                  _vrow(enc["bs"][1]), _vrow(enc["bs"][2]),
                  _vrow(enc["ln_g"]), _vrow(enc["ln_b"]))

    t = jnp.concatenate([pos, mesh_pos, curr_vel, jnp.zeros((n, 2), jnp.float32)], axis=1)
    raw_m, st_m = _edge_feats(_gather_rows(t, src), _gather_rows(t, dst), False)
    raw_w, st_w = _edge_feats(_gather_rows(t, sw), _gather_rows(t, dw), True)

    me = params["mesh_enc"]
    e_m = _encode_edges(raw_m, st_m, me["ws"][0], me["ws"][1], me["ws"][2],
                        _vrow(me["bs"][0]), _vrow(me["bs"][1]), _vrow(me["bs"][2]),
                        _vrow(me["ln_g"]), _vrow(me["ln_b"]))
    we = params["world_enc"]
    w1w = jnp.pad(we["ws"][0], ((0, 4), (0, 0)))
    e_w = _encode_edges(raw_w, st_w, w1w, we["ws"][1], we["ws"][2],
                        _vrow(we["bs"][0]), _vrow(we["bs"][1]), _vrow(we["bs"][2]),
                        _vrow(we["ln_g"]), _vrow(we["ln_b"]))

    blocks = params["blocks"]

    def proj_weights(blk):
        wm1 = blk["mesh"]["ws"][0]
        ww1 = blk["world"]["ws"][0]
        return (wm1[0:HID], wm1[HID:2 * HID], ww1[0:HID], ww1[HID:2 * HID],
                _vrow(blk["mesh"]["bs"][0]), _vrow(blk["world"]["bs"][0]))

    xam, xbm, xaw, xbw = _project(x, *proj_weights(blocks[0]))

    for i, blk in enumerate(blocks):
        bm, bw, bn = blk["mesh"], blk["world"], blk["node"]
        e_m = _edge_update(_gather_rows(xam, src), _gather_rows(xbm, dst), e_m,
                           bm["ws"][0][2 * HID:], bm["ws"][1], bm["ws"][2],
                           _vrow(bm["bs"][1]), _vrow(bm["bs"][2]),
                           _vrow(bm["ln_g"]), _vrow(bm["ln_b"]))
        e_w = _edge_update(_gather_rows(xaw, sw), _gather_rows(xbw, dw), e_w,
                           bw["ws"][0][2 * HID:], bw["ws"][1], bw["ws"][2],
                           _vrow(bw["bs"][1]), _vrow(bw["bs"][2]),
                           _vrow(bw["ln_g"]), _vrow(bw["ln_b"]))
        am0, am1 = _scatter_add(e_m, dst, n)
        aw0, aw1 = _scatter_add(e_w, dw, n)
        wn1 = bn["ws"][0]
        x = _node_update(x, am0, am1, aw0, aw1,
                         wn1[0:HID], wn1[HID:2 * HID], wn1[2 * HID:],
                         _vrow(bn["bs"][0]), bn["ws"][1], bn["ws"][2],
                         _vrow(bn["bs"][1]), _vrow(bn["bs"][2]),
                         _vrow(bn["ln_g"]), _vrow(bn["ln_b"]))
        if i + 1 < len(blocks):
            xam, xbm, xaw, xbw = _project(x, *proj_weights(blocks[i + 1]))

    dec = params["dec"]
    w3d = jnp.pad(dec["ws"][2], ((0, 0), (0, HID - 2)))
    b3d = jnp.pad(dec["bs"][2].reshape(1, 2), ((0, 0), (0, HID - 2)))
    out = _mlp_call(x, dec["ws"][0], dec["ws"][1], w3d, _vrow(dec["bs"][0]),
                    _vrow(dec["bs"][1]), b3d,
                    jnp.ones((1, HID), jnp.float32),
                    jnp.zeros((1, HID), jnp.float32), ln=False)
    return out[:, :2]


# speedup vs baseline: 2.1791x; 1.6528x over previous
import functools

import jax
import jax.numpy as jnp
from jax import lax
from jax.experimental import pallas as pl
from jax.experimental.pallas import tpu as pltpu
from jax.experimental.pallas import tpu_sc as plsc

_pc = pl.pallas_call

HID = 128
BN = 2000
BEDGE = 2048
_NC, _NS = 2, 16
_NW = _NC * _NS
_CH = 128
EPM = 163840
EPW = 20480
NACC = 10240


def _mm(a, b):
    return jax.lax.dot_general(a, b, (((1,), (0,)), ((), ())),
                               preferred_element_type=jnp.float32)


def _ln(h, g, b):
    m = jnp.mean(h, axis=1, keepdims=True)
    v = jnp.mean((h - m) ** 2, axis=1, keepdims=True)
    return (h - m) / jnp.sqrt(v + 1e-5) * g + b


def _row_spec(bs, w=HID):
    return pl.BlockSpec((bs, w), lambda i: (i, 0))


def _full_spec(shape):
    return pl.BlockSpec(shape, lambda i: tuple(0 for _ in shape))


def _sc_gather2(xa, xb, idx_a, idx_b):
    b = idx_a.shape[0]
    width = xa.shape[1]
    per_w = b // _NW
    nch = per_w // _CH
    mesh = plsc.VectorSubcoreMesh(core_axis_name="c", subcore_axis_name="s")

    @functools.partial(
        pl.kernel, mesh=mesh,
        out_type=[jax.ShapeDtypeStruct((b, width), jnp.float32)] * 2,
        scratch_types=[
            pltpu.VMEM((_CH,), jnp.int32), pltpu.VMEM((_CH,), jnp.int32),
            pltpu.VMEM((_CH, width), jnp.float32),
            pltpu.VMEM((_CH, width), jnp.float32),
            pltpu.SemaphoreType.DMA, pltpu.SemaphoreType.DMA,
        ])
    def k(xa_h, xb_h, ia_h, ib_h, ga_h, gb_h, ia_v, ib_v, ra_v, rb_v, sa, sb):
        wid = lax.axis_index("s") * _NC + lax.axis_index("c")
        base = wid * per_w

        def body(c, carry):
            off = base + c * _CH
            pltpu.sync_copy(ia_h.at[pl.ds(off, _CH)], ia_v)
            pltpu.sync_copy(ib_h.at[pl.ds(off, _CH)], ib_v)
            cp_a = pltpu.async_copy(xa_h.at[ia_v], ra_v, sa)
            cp_b = pltpu.async_copy(xb_h.at[ib_v], rb_v, sb)
            cp_a.wait()
            cp_b.wait()
            pltpu.sync_copy(ra_v, ga_h.at[pl.ds(off, _CH)])
            pltpu.sync_copy(rb_v, gb_h.at[pl.ds(off, _CH)])
            return carry

        lax.fori_loop(0, nch, body, 0)

    return k(xa, xb, idx_a, idx_b)


def _sc_scatter_add(e, idx, zeros):
    b, width = e.shape
    per_w = b // _NW
    nch = per_w // _CH
    rows_per_tile = NACC // _NS
    mesh = plsc.VectorSubcoreMesh(core_axis_name="c", subcore_axis_name="s")

    @functools.partial(
        pl.kernel, mesh=mesh,
        out_type=jax.ShapeDtypeStruct((_NC, NACC, width), jnp.float32),
        scratch_types=[
            pltpu.VMEM((_CH,), jnp.int32),
            pltpu.VMEM((_CH, width), jnp.float32),
            pltpu.VMEM_SHARED((NACC, width), jnp.float32),
            pltpu.SemaphoreType.DMA,
        ])
    def k(e_h, idx_h, z_h, out_h, idx_v, rows_v, acc_s, sem):
        cid = lax.axis_index("c")
        sid = lax.axis_index("s")
        wid = sid * _NC + cid

        @pl.when(sid == 0)
        def _():
            pltpu.sync_copy(z_h, acc_s)

        plsc.subcore_barrier()
        base = wid * per_w

        def body(c, carry):
            off = base + c * _CH
            pltpu.sync_copy(idx_h.at[pl.ds(off, _CH)], idx_v)
            pltpu.sync_copy(e_h.at[pl.ds(off, _CH)], rows_v)
            pltpu.sync_copy(rows_v, acc_s.at[idx_v], add=True)
            return carry

        lax.fori_loop(0, nch, body, 0)
        plsc.subcore_barrier()
        r0 = sid * rows_per_tile
        pltpu.sync_copy(acc_s.at[pl.ds(r0, rows_per_tile)],
                        out_h.at[cid, pl.ds(r0, rows_per_tile)])

    return k(e, idx, zeros)


def _edge_upd_body(ga, gb, e, w1c, w2, w3, b2, b3, lng, lnb, out):
    h = jnp.maximum(ga[:] + gb[:] + _mm(e[:], w1c[:]), 0.0)
    h = jnp.maximum(_mm(h, w2[:]) + b2[:], 0.0)
    h = _mm(h, w3[:]) + b3[:]
    out[:] = e[:] + _ln(h, lng[:], lnb[:])


def _edge_update(ga, gb, e, w1c, w2, w3, b2, b3, lng, lnb):
    n = e.shape[0]
    return _pc(
        _edge_upd_body,
        grid=(n // BEDGE,),
        in_specs=[_row_spec(BEDGE)] * 3
        + [_full_spec((HID, HID))] * 3 + [_full_spec((1, HID))] * 4,
        out_specs=_row_spec(BEDGE),
        out_shape=jax.ShapeDtypeStruct((n, HID), jnp.float32),
    )(ga, gb, e, w1c, w2, w3, b2, b3, lng, lnb)


def _node_upd_body(x, am0, am1, aw0, aw1, wx, wm, ww, b1, w2, w3, b2, b3,
                   lng, lnb, out):
    xx = x[:]
    h = (_mm(xx, wx[:]) + _mm(am0[:] + am1[:], wm[:])
         + _mm(aw0[:] + aw1[:], ww[:]) + b1[:])
    h = jnp.maximum(h, 0.0)
    h = jnp.maximum(_mm(h, w2[:]) + b2[:], 0.0)
    h = _mm(h, w3[:]) + b3[:]
    out[:] = xx + _ln(h, lng[:], lnb[:])


def _node_update(x, am0, am1, aw0, aw1, wx, wm, ww, b1, w2, w3, b2, b3,
                 lng, lnb):
    n = x.shape[0]
    return _pc(
        _node_upd_body,
        grid=(n // BN,),
        in_specs=[_row_spec(BN)] * 5
        + [_full_spec((HID, HID))] * 3 + [_full_spec((1, HID))]
        + [_full_spec((HID, HID))] * 2 + [_full_spec((1, HID))] * 4,
        out_specs=_row_spec(BN),
        out_shape=jax.ShapeDtypeStruct((n, HID), jnp.float32),
    )(x, am0, am1, aw0, aw1, wx, wm, ww, b1, w2, w3, b2, b3, lng, lnb)


def _project_body(x, wam, wbm, waw, wbw, b1m, b1w, oam, obm, oaw, obw):
    xx = x[:]
    oam[:] = _mm(xx, wam[:]) + b1m[:]
    obm[:] = _mm(xx, wbm[:])
    oaw[:] = _mm(xx, waw[:]) + b1w[:]
    obw[:] = _mm(xx, wbw[:])


def _project(x, wam, wbm, waw, wbw, b1m, b1w):
    n = x.shape[0]
    sd = jax.ShapeDtypeStruct((n, HID), jnp.float32)
    return _pc(
        _project_body,
        grid=(n // BN,),
        in_specs=[_row_spec(BN)] + [_full_spec((HID, HID))] * 4
        + [_full_spec((1, HID))] * 2,
        out_specs=[_row_spec(BN)] * 4,
        out_shape=[sd, sd, sd, sd],
    )(x, wam, wbm, waw, wbw, b1m, b1w)


def _mlp_body(ln, xin, w1, w2, w3, b1, b2, b3, lng, lnb, out):
    h = jnp.maximum(_mm(xin[:], w1[:]) + b1[:], 0.0)
    h = jnp.maximum(_mm(h, w2[:]) + b2[:], 0.0)
    h = _mm(h, w3[:]) + b3[:]
    out[:] = _ln(h, lng[:], lnb[:]) if ln else h


def _mlp_call(xin, w1, w2, w3, b1, b2, b3, lng, lnb, ln=True):
    n, k = xin.shape
    return _pc(
        functools.partial(_mlp_body, ln),
        grid=(n // BN,),
        in_specs=[_row_spec(BN, k), _full_spec((k, HID)),
                  _full_spec((HID, HID)), _full_spec((HID, HID))]
        + [_full_spec((1, HID))] * 5,
        out_specs=_row_spec(BN),
        out_shape=jax.ShapeDtypeStruct((n, HID), jnp.float32),
    )(xin, w1, w2, w3, b1, b2, b3, lng, lnb)


def _feat_body(world, n_real, ts_ref, td_ref, raw, stats):
    i = pl.program_id(0)
    ts, td = ts_ref[:], td_ref[:]
    rel = ts[:, 0:2] - td[:, 0:2]
    rel_vel = ts[:, 4:6] - td[:, 4:6]
    if world:
        r = jnp.concatenate([rel, rel_vel, jnp.zeros_like(rel),
                             jnp.zeros_like(rel)], axis=1)
    else:
        dist = jnp.sqrt(jnp.sum(rel * rel, axis=1, keepdims=True))
        rel_s = ts[:, 2:4] - td[:, 2:4]
        dist_s = jnp.sqrt(jnp.sum(rel_s * rel_s, axis=1, keepdims=True))
        strain = (dist - dist_s) / (dist_s + 1e-6)
        strain_vec = strain * (rel / dist + 1e-6)
        r = jnp.concatenate([rel, strain_vec, rel_vel, dist, dist_s], axis=1)
    rows = i * BEDGE + lax.broadcasted_iota(jnp.int32, (BEDGE, 1), 0)
    r = jnp.where(rows < n_real, r, 0.0)
    raw[:] = r
    st = jnp.concatenate([jnp.sum(r, axis=0, keepdims=True),
                          jnp.sum(r * r, axis=0, keepdims=True),
                          jnp.zeros((6, 8), jnp.float32)], axis=0)

    @pl.when(i == 0)
    def _():
        stats[:] = st

    @pl.when(i > 0)
    def _():
        stats[:] = stats[:] + st


def _edge_feats(ts, td, world, n_real):
    n = ts.shape[0]
    return _pc(
        functools.partial(_feat_body, world, n_real),
        grid=(n // BEDGE,),
        in_specs=[_row_spec(BEDGE, HID), _row_spec(BEDGE, HID)],
        out_specs=[_row_spec(BEDGE, 8), _full_spec((8, 8))],
        out_shape=[jax.ShapeDtypeStruct((n, 8), jnp.float32),
                   jax.ShapeDtypeStruct((8, 8), jnp.float32)],
    )(ts, td)


def _enc_body(count, raw, stats, w1, w2, w3, b1, b2, b3, lng, lnb, out):
    st = stats[:]
    m = st[0:1, :] * (1.0 / count)
    v = st[1:2, :] * (1.0 / count) - m * m
    norm = (raw[:] - m) / jnp.sqrt(v + 1e-8)
    h = jnp.maximum(_mm(norm, w1[:]) + b1[:], 0.0)
    h = jnp.maximum(_mm(h, w2[:]) + b2[:], 0.0)
    h = _mm(h, w3[:]) + b3[:]
    out[:] = _ln(h, lng[:], lnb[:])


def _encode_edges(raw, stats, n_real, w1, w2, w3, b1, b2, b3, lng, lnb):
    n = raw.shape[0]
    return _pc(
        functools.partial(_enc_body, float(n_real)),
        grid=(n // BEDGE,),
        in_specs=[_row_spec(BEDGE, 8), _full_spec((8, 8)),
                  _full_spec((8, HID)),
                  _full_spec((HID, HID)), _full_spec((HID, HID))]
        + [_full_spec((1, HID))] * 5,
        out_specs=_row_spec(BEDGE),
        out_shape=jax.ShapeDtypeStruct((n, HID), jnp.float32),
    )(raw, stats, w1, w2, w3, b1, b2, b3, lng, lnb)


def _vrow(v):
    return v.reshape(1, HID)


def _symlog(x):
    return jnp.sign(x) * jnp.log1p(jnp.abs(x))


def _znorm_masked(x, mask):
    s = jnp.sum(mask) + 1e-8
    m = jnp.sum(x * mask, axis=0) / s
    v = jnp.sum(((x - m) ** 2) * mask, axis=0) / s
    return (x - m) / jnp.sqrt(v + 1e-8)


def _pad_idx(idx, total, fill):
    return jnp.pad(idx.astype(jnp.int32), (0, total - idx.shape[0]),
                   constant_values=fill)


def kernel(pos, mesh_pos, velocities, inv_mass, node_attr, mask, params,
           edge_index, world_edge_index, batch, face_index):
    n = pos.shape[0]
    em = edge_index.shape[1]
    ew = world_edge_index.shape[1]
    src, dst = edge_index[0], edge_index[1]
    sw, dw = world_edge_index[0], world_edge_index[1]
    curr_vel = velocities[:, 2:]

    src_p = _pad_idx(src, EPM, 0)
    dst_p = _pad_idx(dst, EPM, 0)
    sw_p = _pad_idx(sw, EPW, 0)
    dw_p = _pad_idx(dw, EPW, 0)
    dst_s = _pad_idx(dst, EPM, NACC - 1)
    dw_s = _pad_idx(dw, EPW, NACC - 1)
    zeros_acc = jnp.zeros((NACC, HID), jnp.float32)

    norm_vel = _znorm_masked(velocities, mask)
    inv_m = _znorm_masked(_symlog(inv_mass), mask)
    x9 = jnp.concatenate([norm_vel, inv_m, node_attr], axis=1)
    x16 = jnp.pad(x9, ((0, 0), (0, 7)))

    enc = params["node_enc"]
    w1 = jnp.pad(enc["ws"][0], ((0, 7), (0, 0)))
    x = _mlp_call(x16, w1, enc["ws"][1], enc["ws"][2], _vrow(enc["bs"][0]),
                  _vrow(enc["bs"][1]), _vrow(enc["bs"][2]),
                  _vrow(enc["ln_g"]), _vrow(enc["ln_b"]))

    t = jnp.concatenate([pos, mesh_pos, curr_vel,
                         jnp.zeros((n, HID - 6), jnp.float32)], axis=1)
    ts_m, td_m = _sc_gather2(t, t, src_p, dst_p)
    ts_w, td_w = _sc_gather2(t, t, sw_p, dw_p)
    raw_m, st_m = _edge_feats(ts_m, td_m, False, em)
    raw_w, st_w = _edge_feats(ts_w, td_w, True, ew)

    me = params["mesh_enc"]
    e_m = _encode_edges(raw_m, st_m, em, me["ws"][0], me["ws"][1], me["ws"][2],
                        _vrow(me["bs"][0]), _vrow(me["bs"][1]),
                        _vrow(me["bs"][2]), _vrow(me["ln_g"]), _vrow(me["ln_b"]))
    we = params["world_enc"]
    w1w = jnp.pad(we["ws"][0], ((0, 4), (0, 0)))
    e_w = _encode_edges(raw_w, st_w, ew, w1w, we["ws"][1], we["ws"][2],
                        _vrow(we["bs"][0]), _vrow(we["bs"][1]),
                        _vrow(we["bs"][2]), _vrow(we["ln_g"]), _vrow(we["ln_b"]))

    blocks = params["blocks"]

    def proj_weights(blk):
        wm1 = blk["mesh"]["ws"][0]
        ww1 = blk["world"]["ws"][0]
        return (wm1[0:HID], wm1[HID:2 * HID], ww1[0:HID], ww1[HID:2 * HID],
                _vrow(blk["mesh"]["bs"][0]), _vrow(blk["world"]["bs"][0]))

    xam, xbm, xaw, xbw = _project(x, *proj_weights(blocks[0]))

    for i, blk in enumerate(blocks):
        bm, bw, bn = blk["mesh"], blk["world"], blk["node"]
        ga_m, gb_m = _sc_gather2(xam, xbm, src_p, dst_p)
        e_m = _edge_update(ga_m, gb_m, e_m,
                           bm["ws"][0][2 * HID:], bm["ws"][1], bm["ws"][2],
                           _vrow(bm["bs"][1]), _vrow(bm["bs"][2]),
                           _vrow(bm["ln_g"]), _vrow(bm["ln_b"]))
        ga_w, gb_w = _sc_gather2(xaw, xbw, sw_p, dw_p)
        e_w = _edge_update(ga_w, gb_w, e_w,
                           bw["ws"][0][2 * HID:], bw["ws"][1], bw["ws"][2],
                           _vrow(bw["bs"][1]), _vrow(bw["bs"][2]),
                           _vrow(bw["ln_g"]), _vrow(bw["ln_b"]))
        agg_m = _sc_scatter_add(e_m, dst_s, zeros_acc)
        agg_w = _sc_scatter_add(e_w, dw_s, zeros_acc)
        wn1 = bn["ws"][0]
        x = _node_update(x, agg_m[0, :n], agg_m[1, :n],
                         agg_w[0, :n], agg_w[1, :n],
                         wn1[0:HID], wn1[HID:2 * HID], wn1[2 * HID:],
                         _vrow(bn["bs"][0]), bn["ws"][1], bn["ws"][2],
                         _vrow(bn["bs"][1]), _vrow(bn["bs"][2]),
                         _vrow(bn["ln_g"]), _vrow(bn["ln_b"]))
        if i + 1 < len(blocks):
            xam, xbm, xaw, xbw = _project(x, *proj_weights(blocks[i + 1]))

    dec = params["dec"]
    w3d = jnp.pad(dec["ws"][2], ((0, 0), (0, HID - 2)))
    b3d = jnp.pad(dec["bs"][2].reshape(1, 2), ((0, 0), (0, HID - 2)))
    out = _mlp_call(x, dec["ws"][0], dec["ws"][1], w3d, _vrow(dec["bs"][0]),
                    _vrow(dec["bs"][1]), b3d,
                    jnp.ones((1, HID), jnp.float32),
                    jnp.zeros((1, HID), jnp.float32), ln=False)
    return out[:, :2]


# speedup vs baseline: 2.5892x; 1.1882x over previous
import functools

import jax
import jax.numpy as jnp
from jax import lax
from jax.experimental import pallas as pl
from jax.experimental.pallas import tpu as pltpu
from jax.experimental.pallas import tpu_sc as plsc

_pc = pl.pallas_call

HID = 128
BN = 2000
BEDGE = 2048
_NC, _NS = 2, 16
_NW = _NC * _NS
_CH = 128
EPM = 163840
EPW = 20480
NACC = 10240


def _mm(a, b):
    return jax.lax.dot_general(a, b, (((1,), (0,)), ((), ())),
                               preferred_element_type=jnp.float32)


def _ln(h, g, b):
    m = jnp.mean(h, axis=1, keepdims=True)
    v = jnp.mean((h - m) ** 2, axis=1, keepdims=True)
    return (h - m) / jnp.sqrt(v + 1e-5) * g + b


def _row_spec(bs, w=HID):
    return pl.BlockSpec((bs, w), lambda i: (i, 0))


def _full_spec(shape):
    return pl.BlockSpec(shape, lambda i: tuple(0 for _ in shape))


def _drain(sem, ref, hbm_ref):
    pltpu.make_async_copy(hbm_ref, ref, sem).wait()


def _sc_gather2(xa, xb, idx_a, idx_b):
    width = xa.shape[1]
    nw, nch, ch = idx_a.shape
    b = nw * nch * ch
    per_w = nch * ch
    mesh = plsc.VectorSubcoreMesh(core_axis_name="c", subcore_axis_name="s")

    @functools.partial(
        pl.kernel, mesh=mesh,
        out_type=[jax.ShapeDtypeStruct((b, width), jnp.float32)] * 2,
        scratch_types=[
            pltpu.VMEM((nch, ch), jnp.int32), pltpu.VMEM((nch, ch), jnp.int32),
            pltpu.VMEM((ch, width), jnp.float32),
            pltpu.VMEM((ch, width), jnp.float32),
            pltpu.VMEM((ch, width), jnp.float32),
            pltpu.VMEM((ch, width), jnp.float32),
            pltpu.SemaphoreType.DMA, pltpu.SemaphoreType.DMA,
            pltpu.SemaphoreType.DMA, pltpu.SemaphoreType.DMA,
            pltpu.SemaphoreType.DMA, pltpu.SemaphoreType.DMA,
            pltpu.SemaphoreType.DMA, pltpu.SemaphoreType.DMA,
            pltpu.SemaphoreType.DMA,
        ])
    def k(xa_h, xb_h, ia_h, ib_h, ga_h, gb_h, ia_v, ib_v,
          ra0, rb0, ra1, rb1, sga0, sgb0, sga1, sgb1,
          swa0, swb0, swa1, swb1, sidx):
        wid = lax.axis_index("s") * _NC + lax.axis_index("c")
        base = wid * per_w
        cpi_a = pltpu.async_copy(ia_h.at[wid], ia_v, sidx)
        cpi_b = pltpu.async_copy(ib_h.at[wid], ib_v, sidx)
        cpi_a.wait()
        cpi_b.wait()

        hdum = xa_h.at[pl.ds(0, ch)]

        def fire_gather(c, buf_a, buf_b, sa, sb):
            pltpu.async_copy(xa_h.at[ia_v.at[c]], buf_a, sa)
            pltpu.async_copy(xb_h.at[ib_v.at[c]], buf_b, sb)

        fire_gather(0, ra0, rb0, sga0, sgb0)
        fire_gather(1, ra1, rb1, sga1, sgb1)

        def body(g, carry):
            c0 = 2 * g
            off0 = base + c0 * ch
            off1 = off0 + ch
            _drain(sga0, ra0, hdum)
            _drain(sgb0, rb0, hdum)
            pltpu.async_copy(ra0, ga_h.at[pl.ds(off0, ch)], swa0)
            pltpu.async_copy(rb0, gb_h.at[pl.ds(off0, ch)], swb0)
            _drain(sga1, ra1, hdum)
            _drain(sgb1, rb1, hdum)
            pltpu.async_copy(ra1, ga_h.at[pl.ds(off1, ch)], swa1)
            pltpu.async_copy(rb1, gb_h.at[pl.ds(off1, ch)], swb1)
            _drain(swa0, ra0, hdum)
            _drain(swb0, rb0, hdum)

            @pl.when(c0 + 2 < nch)
            def _():
                fire_gather(c0 + 2, ra0, rb0, sga0, sgb0)

            _drain(swa1, ra1, hdum)
            _drain(swb1, rb1, hdum)

            @pl.when(c0 + 3 < nch)
            def _():
                fire_gather(c0 + 3, ra1, rb1, sga1, sgb1)

            return carry

        lax.fori_loop(0, nch // 2, body, 0)

    return k(xa, xb, idx_a, idx_b)


def _sc_scatter_add(e, idx, zeros):
    b, width = e.shape
    nw, nch, ch = idx.shape
    per_w = nch * ch
    rows_per_tile = NACC // _NS
    mesh = plsc.VectorSubcoreMesh(core_axis_name="c", subcore_axis_name="s")

    @functools.partial(
        pl.kernel, mesh=mesh,
        out_type=jax.ShapeDtypeStruct((_NC, NACC, width), jnp.float32),
        scratch_types=[
            pltpu.VMEM((nch, ch), jnp.int32),
            pltpu.VMEM((ch, width), jnp.float32),
            pltpu.VMEM((ch, width), jnp.float32),
            pltpu.VMEM_SHARED((NACC, width), jnp.float32),
            pltpu.SemaphoreType.DMA, pltpu.SemaphoreType.DMA,
            pltpu.SemaphoreType.DMA, pltpu.SemaphoreType.DMA,
            pltpu.SemaphoreType.DMA,
        ])
    def k(e_h, idx_h, z_h, out_h, idx_v, r0, r1, acc_s,
          sc0, sc1, ss0, ss1, sidx):
        cid = lax.axis_index("c")
        sid = lax.axis_index("s")
        wid = sid * _NC + cid
        base = wid * per_w
        hdum = e_h.at[pl.ds(0, ch)]
        cpi = pltpu.async_copy(idx_h.at[wid], idx_v, sidx)
        pltpu.async_copy(e_h.at[pl.ds(base, ch)], r0, sc0)
        pltpu.async_copy(e_h.at[pl.ds(base + ch, ch)], r1, sc1)

        @pl.when(sid == 0)
        def _():
            pltpu.sync_copy(z_h, acc_s)

        cpi.wait()
        plsc.subcore_barrier()

        def body(g, carry):
            c0 = 2 * g
            _drain(sc0, r0, hdum)
            pltpu.async_copy(r0, acc_s.at[idx_v.at[c0]], ss0, add=True)
            _drain(sc1, r1, hdum)
            pltpu.async_copy(r1, acc_s.at[idx_v.at[c0 + 1]], ss1, add=True)
            _drain(ss0, r0, hdum)

            @pl.when(c0 + 2 < nch)
            def _():
                pltpu.async_copy(e_h.at[pl.ds(base + (c0 + 2) * ch, ch)],
                                 r0, sc0)

            _drain(ss1, r1, hdum)

            @pl.when(c0 + 3 < nch)
            def _():
                pltpu.async_copy(e_h.at[pl.ds(base + (c0 + 3) * ch, ch)],
                                 r1, sc1)

            return carry

        lax.fori_loop(0, nch // 2, body, 0)
        plsc.subcore_barrier()
        r_0 = sid * rows_per_tile
        pltpu.sync_copy(acc_s.at[pl.ds(r_0, rows_per_tile)],
                        out_h.at[cid, pl.ds(r_0, rows_per_tile)])

    return k(e, idx, zeros)


def _edge_upd_body(ga, gb, e, w1c, w2, w3, b2, b3, lng, lnb, out):
    h = jnp.maximum(ga[:] + gb[:] + _mm(e[:], w1c[:]), 0.0)
    h = jnp.maximum(_mm(h, w2[:]) + b2[:], 0.0)
    h = _mm(h, w3[:]) + b3[:]
    out[:] = e[:] + _ln(h, lng[:], lnb[:])


def _edge_update(ga, gb, e, w1c, w2, w3, b2, b3, lng, lnb):
    n = e.shape[0]
    return _pc(
        _edge_upd_body,
        grid=(n // BEDGE,),
        in_specs=[_row_spec(BEDGE)] * 3
        + [_full_spec((HID, HID))] * 3 + [_full_spec((1, HID))] * 4,
        out_specs=_row_spec(BEDGE),
        out_shape=jax.ShapeDtypeStruct((n, HID), jnp.float32),
    )(ga, gb, e, w1c, w2, w3, b2, b3, lng, lnb)


def _node_upd_body(x, am0, am1, aw0, aw1, wx, wm, ww, b1, w2, w3, b2, b3,
                   lng, lnb, out):
    xx = x[:]
    h = (_mm(xx, wx[:]) + _mm(am0[:] + am1[:], wm[:])
         + _mm(aw0[:] + aw1[:], ww[:]) + b1[:])
    h = jnp.maximum(h, 0.0)
    h = jnp.maximum(_mm(h, w2[:]) + b2[:], 0.0)
    h = _mm(h, w3[:]) + b3[:]
    out[:] = xx + _ln(h, lng[:], lnb[:])


def _node_update(x, am0, am1, aw0, aw1, wx, wm, ww, b1, w2, w3, b2, b3,
                 lng, lnb):
    n = x.shape[0]
    return _pc(
        _node_upd_body,
        grid=(n // BN,),
        in_specs=[_row_spec(BN)] * 5
        + [_full_spec((HID, HID))] * 3 + [_full_spec((1, HID))]
        + [_full_spec((HID, HID))] * 2 + [_full_spec((1, HID))] * 4,
        out_specs=_row_spec(BN),
        out_shape=jax.ShapeDtypeStruct((n, HID), jnp.float32),
    )(x, am0, am1, aw0, aw1, wx, wm, ww, b1, w2, w3, b2, b3, lng, lnb)


def _project_body(x, wam, wbm, waw, wbw, b1m, b1w, oam, obm, oaw, obw):
    xx = x[:]
    oam[:] = _mm(xx, wam[:]) + b1m[:]
    obm[:] = _mm(xx, wbm[:])
    oaw[:] = _mm(xx, waw[:]) + b1w[:]
    obw[:] = _mm(xx, wbw[:])


def _project(x, wam, wbm, waw, wbw, b1m, b1w):
    n = x.shape[0]
    sd = jax.ShapeDtypeStruct((n, HID), jnp.float32)
    return _pc(
        _project_body,
        grid=(n // BN,),
        in_specs=[_row_spec(BN)] + [_full_spec((HID, HID))] * 4
        + [_full_spec((1, HID))] * 2,
        out_specs=[_row_spec(BN)] * 4,
        out_shape=[sd, sd, sd, sd],
    )(x, wam, wbm, waw, wbw, b1m, b1w)


def _mlp_body(ln, xin, w1, w2, w3, b1, b2, b3, lng, lnb, out):
    h = jnp.maximum(_mm(xin[:], w1[:]) + b1[:], 0.0)
    h = jnp.maximum(_mm(h, w2[:]) + b2[:], 0.0)
    h = _mm(h, w3[:]) + b3[:]
    out[:] = _ln(h, lng[:], lnb[:]) if ln else h


def _mlp_call(xin, w1, w2, w3, b1, b2, b3, lng, lnb, ln=True):
    n, k = xin.shape
    return _pc(
        functools.partial(_mlp_body, ln),
        grid=(n // BN,),
        in_specs=[_row_spec(BN, k), _full_spec((k, HID)),
                  _full_spec((HID, HID)), _full_spec((HID, HID))]
        + [_full_spec((1, HID))] * 5,
        out_specs=_row_spec(BN),
        out_shape=jax.ShapeDtypeStruct((n, HID), jnp.float32),
    )(xin, w1, w2, w3, b1, b2, b3, lng, lnb)


def _feat_body(world, n_real, ts_ref, td_ref, raw, stats):
    i = pl.program_id(0)
    ts, td = ts_ref[:], td_ref[:]
    rel = ts[:, 0:2] - td[:, 0:2]
    rel_vel = ts[:, 4:6] - td[:, 4:6]
    if world:
        r = jnp.concatenate([rel, rel_vel, jnp.zeros_like(rel),
                             jnp.zeros_like(rel)], axis=1)
    else:
        dist = jnp.sqrt(jnp.sum(rel * rel, axis=1, keepdims=True))
        rel_s = ts[:, 2:4] - td[:, 2:4]
        dist_s = jnp.sqrt(jnp.sum(rel_s * rel_s, axis=1, keepdims=True))
        strain = (dist - dist_s) / (dist_s + 1e-6)
        strain_vec = strain * (rel / dist + 1e-6)
        r = jnp.concatenate([rel, strain_vec, rel_vel, dist, dist_s], axis=1)
    rows = i * BEDGE + lax.broadcasted_iota(jnp.int32, (BEDGE, 1), 0)
    r = jnp.where(rows < n_real, r, 0.0)
    raw[:] = r
    st = jnp.concatenate([jnp.sum(r, axis=0, keepdims=True),
                          jnp.sum(r * r, axis=0, keepdims=True),
                          jnp.zeros((6, 8), jnp.float32)], axis=0)

    @pl.when(i == 0)
    def _():
        stats[:] = st

    @pl.when(i > 0)
    def _():
        stats[:] = stats[:] + st


def _edge_feats(ts, td, world, n_real):
    n = ts.shape[0]
    return _pc(
        functools.partial(_feat_body, world, n_real),
        grid=(n // BEDGE,),
        in_specs=[_row_spec(BEDGE, HID), _row_spec(BEDGE, HID)],
        out_specs=[_row_spec(BEDGE, 8), _full_spec((8, 8))],
        out_shape=[jax.ShapeDtypeStruct((n, 8), jnp.float32),
                   jax.ShapeDtypeStruct((8, 8), jnp.float32)],
    )(ts, td)


def _enc_body(count, raw, stats, w1, w2, w3, b1, b2, b3, lng, lnb, out):
    st = stats[:]
    m = st[0:1, :] * (1.0 / count)
    v = st[1:2, :] * (1.0 / count) - m * m
    norm = (raw[:] - m) / jnp.sqrt(v + 1e-8)
    h = jnp.maximum(_mm(norm, w1[:]) + b1[:], 0.0)
    h = jnp.maximum(_mm(h, w2[:]) + b2[:], 0.0)
    h = _mm(h, w3[:]) + b3[:]
    out[:] = _ln(h, lng[:], lnb[:])


def _encode_edges(raw, stats, n_real, w1, w2, w3, b1, b2, b3, lng, lnb):
    n = raw.shape[0]
    return _pc(
        functools.partial(_enc_body, float(n_real)),
        grid=(n // BEDGE,),
        in_specs=[_row_spec(BEDGE, 8), _full_spec((8, 8)),
                  _full_spec((8, HID)),
                  _full_spec((HID, HID)), _full_spec((HID, HID))]
        + [_full_spec((1, HID))] * 5,
        out_specs=_row_spec(BEDGE),
        out_shape=jax.ShapeDtypeStruct((n, HID), jnp.float32),
    )(raw, stats, w1, w2, w3, b1, b2, b3, lng, lnb)


def _vrow(v):
    return v.reshape(1, HID)


def _symlog(x):
    return jnp.sign(x) * jnp.log1p(jnp.abs(x))


def _znorm_masked(x, mask):
    s = jnp.sum(mask) + 1e-8
    m = jnp.sum(x * mask, axis=0) / s
    v = jnp.sum(((x - m) ** 2) * mask, axis=0) / s
    return (x - m) / jnp.sqrt(v + 1e-8)


def _pad_idx(idx, total, fill, ch):
    p = jnp.pad(idx.astype(jnp.int32), (0, total - idx.shape[0]),
                constant_values=fill)
    return p.reshape(_NW, total // (_NW * ch), ch)


def kernel(pos, mesh_pos, velocities, inv_mass, node_attr, mask, params,
           edge_index, world_edge_index, batch, face_index):
    n = pos.shape[0]
    em = edge_index.shape[1]
    ew = world_edge_index.shape[1]
    src, dst = edge_index[0], edge_index[1]
    sw, dw = world_edge_index[0], world_edge_index[1]
    curr_vel = velocities[:, 2:]

    src_p = _pad_idx(src, EPM, 0, 128)
    dst_p = _pad_idx(dst, EPM, 0, 128)
    sw_p = _pad_idx(sw, EPW, 0, 64)
    dw_p = _pad_idx(dw, EPW, 0, 64)
    dst_s = _pad_idx(dst, EPM, NACC - 1, 128)
    dw_s = _pad_idx(dw, EPW, NACC - 1, 64)
    zeros_acc = jnp.zeros((NACC, HID), jnp.float32)

    norm_vel = _znorm_masked(velocities, mask)
    inv_m = _znorm_masked(_symlog(inv_mass), mask)
    x9 = jnp.concatenate([norm_vel, inv_m, node_attr], axis=1)
    x16 = jnp.pad(x9, ((0, 0), (0, 7)))

    enc = params["node_enc"]
    w1 = jnp.pad(enc["ws"][0], ((0, 7), (0, 0)))
    x = _mlp_call(x16, w1, enc["ws"][1], enc["ws"][2], _vrow(enc["bs"][0]),
                  _vrow(enc["bs"][1]), _vrow(enc["bs"][2]),
                  _vrow(enc["ln_g"]), _vrow(enc["ln_b"]))

    t = jnp.concatenate([pos, mesh_pos, curr_vel,
                         jnp.zeros((n, HID - 6), jnp.float32)], axis=1)
    ts_m, td_m = _sc_gather2(t, t, src_p, dst_p)
    ts_w, td_w = _sc_gather2(t, t, sw_p, dw_p)
    raw_m, st_m = _edge_feats(ts_m, td_m, False, em)
    raw_w, st_w = _edge_feats(ts_w, td_w, True, ew)

    me = params["mesh_enc"]
    e_m = _encode_edges(raw_m, st_m, em, me["ws"][0], me["ws"][1], me["ws"][2],
                        _vrow(me["bs"][0]), _vrow(me["bs"][1]),
                        _vrow(me["bs"][2]), _vrow(me["ln_g"]), _vrow(me["ln_b"]))
    we = params["world_enc"]
    w1w = jnp.pad(we["ws"][0], ((0, 4), (0, 0)))
    e_w = _encode_edges(raw_w, st_w, ew, w1w, we["ws"][1], we["ws"][2],
                        _vrow(we["bs"][0]), _vrow(we["bs"][1]),
                        _vrow(we["bs"][2]), _vrow(we["ln_g"]), _vrow(we["ln_b"]))

    blocks = params["blocks"]

    def proj_weights(blk):
        wm1 = blk["mesh"]["ws"][0]
        ww1 = blk["world"]["ws"][0]
        return (wm1[0:HID], wm1[HID:2 * HID], ww1[0:HID], ww1[HID:2 * HID],
                _vrow(blk["mesh"]["bs"][0]), _vrow(blk["world"]["bs"][0]))

    xam, xbm, xaw, xbw = _project(x, *proj_weights(blocks[0]))

    for i, blk in enumerate(blocks):
        bm, bw, bn = blk["mesh"], blk["world"], blk["node"]
        ga_m, gb_m = _sc_gather2(xam, xbm, src_p, dst_p)
        e_m = _edge_update(ga_m, gb_m, e_m,
                           bm["ws"][0][2 * HID:], bm["ws"][1], bm["ws"][2],
                           _vrow(bm["bs"][1]), _vrow(bm["bs"][2]),
                           _vrow(bm["ln_g"]), _vrow(bm["ln_b"]))
        ga_w, gb_w = _sc_gather2(xaw, xbw, sw_p, dw_p)
        e_w = _edge_update(ga_w, gb_w, e_w,
                           bw["ws"][0][2 * HID:], bw["ws"][1], bw["ws"][2],
                           _vrow(bw["bs"][1]), _vrow(bw["bs"][2]),
                           _vrow(bw["ln_g"]), _vrow(bw["ln_b"]))
        agg_m = _sc_scatter_add(e_m, dst_s, zeros_acc)
        agg_w = _sc_scatter_add(e_w, dw_s, zeros_acc)
        wn1 = bn["ws"][0]
        x = _node_update(x, agg_m[0, :n], agg_m[1, :n],
                         agg_w[0, :n], agg_w[1, :n],
                         wn1[0:HID], wn1[HID:2 * HID], wn1[2 * HID:],
                         _vrow(bn["bs"][0]), bn["ws"][1], bn["ws"][2],
                         _vrow(bn["bs"][1]), _vrow(bn["bs"][2]),
                         _vrow(bn["ln_g"]), _vrow(bn["ln_b"]))
        if i + 1 < len(blocks):
            xam, xbm, xaw, xbw = _project(x, *proj_weights(blocks[i + 1]))

    dec = params["dec"]
    w3d = jnp.pad(dec["ws"][2], ((0, 0), (0, HID - 2)))
    b3d = jnp.pad(dec["bs"][2].reshape(1, 2), ((0, 0), (0, HID - 2)))
    out = _mlp_call(x, dec["ws"][0], dec["ws"][1], w3d, _vrow(dec["bs"][0]),
                    _vrow(dec["bs"][1]), b3d,
                    jnp.ones((1, HID), jnp.float32),
                    jnp.zeros((1, HID), jnp.float32), ln=False)
    return out[:, :2]
